# baseline (speedup 1.0000x reference)
import dataclasses
import functools

import jax
import jax.numpy as jnp
from jax import lax
from jax.experimental import pallas as pl
from jax.experimental.pallas import tpu as pltpu
from jax.experimental.pallas import tpu_sc as plsc

B = 4
S = 2048
D = 768
N = B * S
PAD = 1
EPS = 1e-5
NC = 2
NS = 16
NW = NC * NS
TPW = N // NW
K = 64
NSUB = TPW // K
CPR = S // TPW
DV = D // 16


def _sc_body(ids_hbm, tti_hbm, word_hbm, pos_hbm, tte_hbm, g_hbm, b_hbm,
             out_hbm,
             ids_row, pos_row, tti_vm, ids2, pos2, tte_v, g_v, b_v,
             bufA, bufB):
    wid = lax.axis_index("s") * NC + lax.axis_index("c")
    row = wid // CPR
    chunk = wid % CPR
    row_base = row * S
    chunk_off = chunk * TPW
    tok_base = row_base + chunk_off

    pltpu.sync_copy(ids_hbm.at[pl.ds(row_base, S)], ids_row)
    pltpu.sync_copy(tti_hbm.at[pl.ds(tok_base, TPW)], tti_vm.at[pl.ds(0, TPW)])
    pltpu.sync_copy(tte_hbm, tte_v)
    pltpu.sync_copy(g_hbm, g_v)
    pltpu.sync_copy(b_hbm, b_v)

    def pos_step(i, carry):
        v = ids_row[pl.ds(i * 16, 16)]
        m = (v != PAD).astype(jnp.int32)
        cs = jnp.cumsum(m) + carry
        pos_row[pl.ds(i * 16, 16)] = cs * m + PAD
        return carry + jnp.sum(m)

    lax.fori_loop(0, S // 16, pos_step, jnp.int32(0))

    @pl.loop(0, NSUB)
    def _(j):
        @pl.loop(0, K // 16)
        def _(i):
            src = chunk_off + j * K + i * 16
            ids2[j, pl.ds(i * 16, 16)] = ids_row[pl.ds(src, 16)]
            pos2[j, pl.ds(i * 16, 16)] = pos_row[pl.ds(src, 16)]

    @pl.loop(0, NSUB)
    def _(j):
        pltpu.sync_copy(word_hbm.at[ids2.at[j]], bufA)
        pltpu.sync_copy(pos_hbm.at[pos2.at[j]], bufB)

        @pl.loop(0, K)
        def _(t):
            tvec = tti_vm[pl.ds(j * K + t, 16)]
            tbase = tvec[0] * D
            acc = jnp.zeros((16,), jnp.float32)
            acc2 = jnp.zeros((16,), jnp.float32)
            xs = []
            for d in range(DV):
                x = (bufA[t, pl.ds(d * 16, 16)]
                     + bufB[t, pl.ds(d * 16, 16)]
                     + tte_v[pl.ds(tbase + d * 16, 16)])
                xs.append(x)
                acc = acc + x
                acc2 = acc2 + x * x
            mean = jnp.sum(acc) * (1.0 / D)
            var = jnp.sum(acc2) * (1.0 / D) - mean * mean
            ve = jnp.full((16,), var + EPS, dtype=jnp.float32)
            yi = plsc.bitcast(ve, jnp.int32)
            yi = 0x5F3759DF - lax.shift_right_logical(yi, 1)
            r = plsc.bitcast(yi, jnp.float32)
            half = ve * 0.5
            for _ in range(3):
                r = r * (1.5 - half * r * r)
            meanv = jnp.full((16,), mean, dtype=jnp.float32)
            for d in range(DV):
                y = ((xs[d] - meanv) * r * g_v[pl.ds(d * 16, 16)]
                     + b_v[pl.ds(d * 16, 16)])
                bufA[t, pl.ds(d * 16, 16)] = y

        pltpu.sync_copy(bufA, out_hbm.at[pl.ds(tok_base + j * K, K)])


@jax.jit
def _sc_call(ids, tti, word, pos, tte_flat, gamma, beta):
    mesh = plsc.VectorSubcoreMesh(core_axis_name="c", subcore_axis_name="s")
    cp = pltpu.CompilerParams()
    if "needs_layout_passes" in pltpu.CompilerParams.__dataclass_fields__:
        cp = dataclasses.replace(cp, needs_layout_passes=False)
    f = functools.partial(
        pl.kernel,
        out_type=jax.ShapeDtypeStruct((N, D), jnp.float32),
        mesh=mesh,
        compiler_params=cp,
        scratch_types=[
            pltpu.VMEM((S,), jnp.int32),
            pltpu.VMEM((S,), jnp.int32),
            pltpu.VMEM((TPW + 16,), jnp.int32),
            pltpu.VMEM((NSUB, K), jnp.int32),
            pltpu.VMEM((NSUB, K), jnp.int32),
            pltpu.VMEM((2 * D,), jnp.float32),
            pltpu.VMEM((D,), jnp.float32),
            pltpu.VMEM((D,), jnp.float32),
            pltpu.VMEM((K, D), jnp.float32),
            pltpu.VMEM((K, D), jnp.float32),
        ],
    )(_sc_body)
    return f(ids, tti, word, pos, tte_flat, gamma, beta)


def kernel(input_ids, token_type_ids, word_embeddings, position_embeddings,
           token_type_embeddings, ln_gamma, ln_beta):
    ids = input_ids.reshape(-1).astype(jnp.int32)
    tti = token_type_ids.reshape(-1).astype(jnp.int32)
    tte_flat = token_type_embeddings.reshape(-1)
    out = _sc_call(ids, tti, word_embeddings, position_embeddings, tte_flat,
                   ln_gamma, ln_beta)
    return out.reshape(input_ids.shape[0], input_ids.shape[1], D)
